# Initial kernel scaffold; baseline (speedup 1.0000x reference)
import jax
import jax.numpy as jnp
from jax.experimental import pallas as pl


def kernel(x, prototypes, background):
    raise NotImplementedError("write your pallas kernel here")


# speedup vs baseline: 4.5448x; 4.5448x over previous
import numpy as np
import jax
import jax.numpy as jnp
from jax.experimental import pallas as pl
from jax.experimental.pallas import tpu as pltpu

P = 6
L = 4
H = 64
W = 64
NCAND = P * L + 1

_k = np.arange(H)
_ang = -2.0 * np.pi * np.outer(_k, _k) / H
_DR = np.cos(_ang).astype(np.float32)
_DI = np.sin(_ang).astype(np.float32)

_HIGH = jax.lax.Precision.HIGHEST


def _mm(a, b):
    return jnp.dot(a, b, precision=_HIGH, preferred_element_type=jnp.float32)


def _overwrite(rec, obj):
    m = (obj > 0.1).astype(jnp.float32)
    return rec * (1.0 - m) + obj * m


def _proto_fft_kernel(protos_ref, dr_ref, di_ref, fpr_ref, fpi_ref):
    dr = dr_ref[...]
    di = di_ref[...]
    for p in range(P):
        pr = protos_ref[p]
        ar = _mm(dr, pr)
        ai = _mm(di, pr)
        fpr_ref[p] = _mm(ar, dr.T) - _mm(ai, di.T)
        fpi_ref[p] = _mm(ar, di.T) + _mm(ai, dr.T)


def _main_kernel(x_ref, fpr_ref, fpi_ref, protos_ref, bg_ref, dr_ref, di_ref,
                 out_ref, temps_ref):
    xb = x_ref[0]
    dr = dr_ref[...]
    di = di_ref[...]
    bg = bg_ref[...]

    ar = _mm(dr, xb)
    ai = _mm(di, xb)
    fxr = _mm(ar, dr.T) - _mm(ai, di.T)
    fxi = _mm(ar, di.T) + _mm(ai, dr.T)

    er = dr * (1.0 / H)
    ei = di * (-1.0 / H)

    lin = (jax.lax.broadcasted_iota(jnp.int32, (H, W), 0) * W
           + jax.lax.broadcasted_iota(jnp.int32, (H, W), 1))

    temps_ref[0] = jnp.zeros((H, W), jnp.float32)

    for p in range(P):
        fpr = fpr_ref[p]
        fpi = fpi_ref[p]
        cr = fxr * fpr + fxi * fpi
        ci = fxi * fpr - fxr * fpi
        den = jnp.sqrt(cr * cr + ci * ci) + 1e-8
        cr = cr / den
        ci = ci / den
        gr = _mm(er, cr) - _mm(ei, ci)
        gi = _mm(er, ci) + _mm(ei, cr)
        corr = _mm(gr, er.T) - _mm(gi, ei.T)

        proto = protos_ref[p]
        for l in range(L):
            m = jnp.max(corr)
            idx = jnp.min(jnp.where(corr == m, lin, jnp.int32(1 << 30)))
            corr = jnp.where(lin == idx, jnp.float32(-3.0e38), corr)
            r = idx // W
            c = idx % W
            t = proto
            for bit in range(6):
                amt = 1 << bit
                t = jnp.where(((r >> bit) & 1) == 1,
                              jnp.roll(t, amt, axis=0), t)
                t = jnp.where(((c >> bit) & 1) == 1,
                              jnp.roll(t, amt, axis=1), t)
            temps_ref[1 + p * L + l] = t

    aux = []
    used = [jnp.zeros((), jnp.bool_) for _ in range(NCAND)]
    inv_n = jnp.float32(1.0 / (H * W))
    for l in range(L):
        best_val = None
        best_idx = None
        for i in range(NCAND):
            t = temps_ref[i]
            rec = _overwrite(bg, t)
            for j in reversed(range(l)):
                rec = _overwrite(rec, aux[j])
            d = xb - rec
            e = jnp.sum(d * d) * inv_n
            if i > 0:
                e = jnp.where(used[i], jnp.float32(1e8), e)
            if best_val is None:
                best_val = e
                best_idx = jnp.zeros((), jnp.int32)
            else:
                take = e < best_val
                best_idx = jnp.where(take, jnp.int32(i), best_idx)
                best_val = jnp.where(take, e, best_val)
        sel = best_idx
        for i in range(1, NCAND):
            used[i] = jnp.logical_or(used[i], sel == i)
        aux.append(temps_ref[sel])

    rec = bg
    for j in reversed(range(L)):
        rec = _overwrite(rec, aux[j])
    out_ref[0] = rec


def kernel(x, prototypes, background):
    B = x.shape[0]
    x2 = x.reshape(B, H, W)
    protos = prototypes.reshape(P, H, W)
    bg = background.reshape(H, W)
    dr = jnp.asarray(_DR)
    di = jnp.asarray(_DI)

    fpr, fpi = pl.pallas_call(
        _proto_fft_kernel,
        out_shape=[
            jax.ShapeDtypeStruct((P, H, W), jnp.float32),
            jax.ShapeDtypeStruct((P, H, W), jnp.float32),
        ],
    )(protos, dr, di)

    out = pl.pallas_call(
        _main_kernel,
        grid=(B,),
        in_specs=[
            pl.BlockSpec((1, H, W), lambda b: (b, 0, 0)),
            pl.BlockSpec((P, H, W), lambda b: (0, 0, 0)),
            pl.BlockSpec((P, H, W), lambda b: (0, 0, 0)),
            pl.BlockSpec((P, H, W), lambda b: (0, 0, 0)),
            pl.BlockSpec((H, W), lambda b: (0, 0)),
            pl.BlockSpec((H, W), lambda b: (0, 0)),
            pl.BlockSpec((H, W), lambda b: (0, 0)),
        ],
        out_specs=pl.BlockSpec((1, H, W), lambda b: (b, 0, 0)),
        out_shape=jax.ShapeDtypeStruct((B, H, W), jnp.float32),
        scratch_shapes=[pltpu.VMEM((NCAND, H, W), jnp.float32)],
    )(x2, fpr, fpi, protos, bg, dr, di)

    return out.reshape(B, 1, H, W)


# speedup vs baseline: 17.9849x; 3.9573x over previous
import numpy as np
import jax
import jax.numpy as jnp
from jax.experimental import pallas as pl
from jax.experimental.pallas import tpu as pltpu

P = 6
L = 4
H = 64
W = 64
B = 16
NIMG = B + P
NPAIR = B * P
NCAND = P * L + 1

_k = np.arange(H)
_ang = -2.0 * np.pi * np.outer(_k, _k) / H
_DR = np.cos(_ang).astype(np.float32)
_DI = np.sin(_ang).astype(np.float32)

_HIGH = jax.lax.Precision.HIGHEST


def _mm(a, b):
    return jnp.dot(a, b, precision=_HIGH, preferred_element_type=jnp.float32)


def _corr_kernel(x_ref, protos_ref, dr_ref, di_ref, idx_ref):
    dr = dr_ref[...]
    di = di_ref[...]

    imgs = jnp.concatenate([x_ref[...], protos_ref[...]], axis=0)
    imf = imgs.reshape(NIMG * H, W)
    yr = _mm(imf, dr.T)
    yi = _mm(imf, di.T)
    yr_l = yr.reshape(NIMG, H, W).transpose(1, 0, 2).reshape(H, NIMG * W)
    yi_l = yi.reshape(NIMG, H, W).transpose(1, 0, 2).reshape(H, NIMG * W)
    yy = jnp.concatenate([yr_l, yi_l], axis=0)
    l1 = jnp.concatenate([dr, -di], axis=1)
    l2 = jnp.concatenate([di, dr], axis=1)
    fr = _mm(l1, yy)
    fi = _mm(l2, yy)
    fxr, fpr = fr[:, :B * W], fr[:, B * W:]
    fxi, fpi = fi[:, :B * W], fi[:, B * W:]

    def _expand_x(a):
        return jnp.broadcast_to(a.reshape(H, B, 1, W),
                                (H, B, P, W)).reshape(H, NPAIR * W)

    def _expand_p(a):
        return jnp.broadcast_to(a.reshape(H, 1, P, W),
                                (H, B, P, W)).reshape(H, NPAIR * W)

    exr, exi = _expand_x(fxr), _expand_x(fxi)
    epr, epi = _expand_p(fpr), _expand_p(fpi)
    cr = exr * epr + exi * epi
    ci = exi * epr - exr * epi
    den = jnp.sqrt(cr * cr + ci * ci) + 1e-8
    cr = cr / den
    ci = ci / den

    er = dr * (1.0 / H)
    ei = di * (-1.0 / H)
    cc = jnp.concatenate([cr, ci], axis=0)
    e1 = jnp.concatenate([er, -ei], axis=1)
    e2 = jnp.concatenate([ei, er], axis=1)
    gr = _mm(e1, cc)
    gi = _mm(e2, cc)
    gr_r = gr.reshape(H, NPAIR, W).transpose(1, 0, 2).reshape(NPAIR * H, W)
    gi_r = gi.reshape(H, NPAIR, W).transpose(1, 0, 2).reshape(NPAIR * H, W)
    gg = jnp.concatenate([gr_r, gi_r], axis=1)
    et = jnp.concatenate([er.T, -ei.T], axis=0)
    corr_rows = _mm(gg, et)
    corr = corr_rows.reshape(NPAIR, H, W)

    lin = (jax.lax.broadcasted_iota(jnp.int32, (1, H, W), 1) * W
           + jax.lax.broadcasted_iota(jnp.int32, (1, H, W), 2))
    cols = []
    work = corr
    for _ in range(L):
        m = jnp.max(work, axis=(1, 2), keepdims=True)
        ii = jnp.min(jnp.where(work == m, lin, jnp.int32(1 << 30)),
                     axis=(1, 2), keepdims=True)
        cols.append(ii.reshape(NPAIR, 1))
        work = jnp.where(lin == ii, jnp.float32(-3.0e38), work)

    idx_ref[...] = jnp.concatenate(cols, axis=1)


def _select_kernel(idx_ref, x_ref, protos_ref, bg_ref, out_ref, temps_ref):
    temps_ref[:, 0] = jnp.zeros((B, H, W), jnp.float32)
    for p in range(P):
        proto = protos_ref[p]
        for b in range(B):
            for t in range(L):
                ii = idx_ref[b * P + p, t]
                r = ii // W
                c = ii % W
                rolled = pltpu.roll(pltpu.roll(proto, r, 0), c, 1)
                temps_ref[b, 1 + p * L + t] = rolled

    x_all = x_ref[...]
    bg = bg_ref[...]
    temps = temps_ref[...]
    inv_n = jnp.float32(1.0 / (H * W))

    mask_t = temps > 0.1
    xb2 = ((x_all - bg[None]) ** 2)[:, None]
    q = jnp.where(mask_t, (x_all[:, None] - temps) ** 2, xb2)
    e = jnp.sum(q, axis=(2, 3)) * inv_n

    cidx = jax.lax.broadcasted_iota(jnp.int32, (B, NCAND), 1)
    used = jnp.zeros((B, NCAND), jnp.bool_)
    ucov = jnp.zeros((B, H, W), jnp.bool_)
    vals = jnp.zeros((B, H, W), jnp.float32)

    for l in range(L):
        em = jnp.where(used, jnp.float32(1e8), e)
        mval = jnp.min(em, axis=1, keepdims=True)
        sel = jnp.min(jnp.where(em == mval, cidx, jnp.int32(NCAND)),
                      axis=1, keepdims=True)
        used = used | ((cidx == sel) & (sel > 0))
        chosen = jnp.concatenate(
            [temps_ref[b, sel[b, 0]][None] for b in range(B)], axis=0)
        mnew = chosen > 0.1
        add = mnew & ~ucov
        vals = jnp.where(add, chosen, vals)
        ucov = ucov | mnew
        if l + 1 < L:
            addf = add[:, None].astype(jnp.float32)
            e = e - jnp.sum(q * addf, axis=(2, 3)) * inv_n

    out_ref[...] = jnp.where(ucov, vals, bg[None])


def kernel(x, prototypes, background):
    x2 = x.reshape(B, H, W)
    protos = prototypes.reshape(P, H, W)
    bg = background.reshape(H, W)
    dr = jnp.asarray(_DR)
    di = jnp.asarray(_DI)

    idx = pl.pallas_call(
        _corr_kernel,
        out_shape=jax.ShapeDtypeStruct((NPAIR, L), jnp.int32),
    )(x2, protos, dr, di)

    out = pl.pallas_call(
        _select_kernel,
        in_specs=[
            pl.BlockSpec(memory_space=pltpu.SMEM),
            pl.BlockSpec((B, H, W), lambda: (0, 0, 0)),
            pl.BlockSpec((P, H, W), lambda: (0, 0, 0)),
            pl.BlockSpec((H, W), lambda: (0, 0)),
        ],
        out_shape=jax.ShapeDtypeStruct((B, H, W), jnp.float32),
        scratch_shapes=[
            pltpu.VMEM((B, NCAND, H, W), jnp.float32),
        ],
    )(idx, x2, protos, bg)

    return out.reshape(B, 1, H, W)
